# baseline (speedup 1.0000x reference)
import functools

import jax
import jax.numpy as jnp
from jax import lax
from jax.experimental import pallas as pl
from jax.experimental.pallas import tpu as pltpu
from jax.experimental.pallas import tpu_sc as plsc

NUM_CLASSES = 1000000
NUM_SAMPLED = 128
BATCH = 4096
D_MODEL = 64

NC, NS, L = 2, 16, 16
NW = NC * NS
BPW = BATCH // NW
C = 2
NCHUNK = BPW // C
NGRP = NUM_SAMPLED // L
TPAIR = NUM_CLASSES // 2

_mesh = plsc.VectorSubcoreMesh(
    core_axis_name="c", subcore_axis_name="s", num_cores=NC, num_subcores=NS)


@functools.partial(
    pl.kernel,
    out_type=(
        jax.ShapeDtypeStruct((BATCH,), jnp.float32),
        jax.ShapeDtypeStruct((BATCH,), jnp.float32),
    ),
    mesh=_mesh,
    scratch_types=(
        pltpu.VMEM((BPW, D_MODEL), jnp.float32),
        pltpu.VMEM((BPW, 2 * D_MODEL), jnp.float32),
        pltpu.VMEM((NCHUNK, C * NUM_SAMPLED), jnp.int32),
        pltpu.VMEM((BPW,), jnp.int32),
        pltpu.VMEM((C * NUM_SAMPLED, 2 * D_MODEL), jnp.float32),
        pltpu.VMEM((C * NUM_SAMPLED, 2 * D_MODEL), jnp.float32),
        pltpu.VMEM((BPW,), jnp.float32),
        pltpu.VMEM((BPW,), jnp.float32),
        pltpu.VMEM((BPW,), jnp.float32),
        pltpu.SemaphoreType.DMA,
        pltpu.SemaphoreType.DMA,
        pltpu.SemaphoreType.DMA,
    ),
    compiler_params=pltpu.CompilerParams(needs_layout_passes=False,
                                         use_tc_tiling_on_sc=True),
)
def _sc_sampled_softmax(user_hbm, table_hbm, posid_hbm, negid_hbm,
                        se_out, mp_out,
                        user_v, posrows_v, negids_v, posids_v,
                        nbuf_a, nbuf_b, pos_v, se_v, mp_v,
                        sem_p, sem_a, sem_b):
    wid = lax.axis_index("s") * NC + lax.axis_index("c")
    base = wid * BPW
    iot = lax.iota(jnp.int32, L)
    lane0 = iot == 0

    pltpu.sync_copy(posid_hbm.at[pl.ds(base, BPW)], posids_v)
    pltpu.sync_copy(negid_hbm.at[wid], negids_v)
    pltpu.sync_copy(user_hbm.at[pl.ds(base, BPW)], user_v)

    def _issue(c, nbuf, sem):
        for k in range(C * NUM_SAMPLED // L):
            idxv = negids_v[c, pl.ds(k * L, L)] >> 1
            pltpu.async_copy(table_hbm.at[idxv],
                             nbuf.at[pl.ds(k * L, L)], sem)

    def _drain(c, nbuf, sem):
        for k in range(C * NUM_SAMPLED // L):
            pltpu.make_async_copy(table_hbm.at[pl.ds(0, L)],
                                  nbuf.at[pl.ds(k * L, L)], sem).wait()

    for k in range(BPW // L):
        pidxv = posids_v[pl.ds(k * L, L)] >> 1
        pltpu.async_copy(table_hbm.at[pidxv],
                         posrows_v.at[pl.ds(k * L, L)], sem_p)
    _issue(0, nbuf_a, sem_a)
    _issue(1, nbuf_b, sem_b)
    for k in range(BPW // L):
        pltpu.make_async_copy(table_hbm.at[pl.ds(0, L)],
                              posrows_v.at[pl.ds(k * L, L)], sem_p).wait()

    for grp in range(BPW // L):
        rows = grp * L + iot
        podd = (posids_v[pl.ds(grp * L, L)] & 1) * D_MODEL

        @pl.loop(0, D_MODEL, init_carry=jnp.zeros((L,), jnp.float32), unroll=4)
        def _pos_dot(d, acc, rows=rows, podd=podd):
            t = d & (L - 1)
            dcol = (d - t) + ((t + iot) & (L - 1))
            u = plsc.load_gather(user_v, [rows, dcol])
            p = plsc.load_gather(posrows_v, [rows, podd + dcol])
            return acc + u * p

        pos_v[pl.ds(grp * L, L)] = _pos_dot

    zero8 = tuple(jnp.zeros((L,), jnp.float32) for _ in range(NGRP))

    @pl.loop(0, NCHUNK, step=2)
    def _chunks(g):
        for buf, (nbuf, sem) in enumerate(((nbuf_a, sem_a), (nbuf_b, sem_b))):
            cidx = g + buf
            _drain(cidx, nbuf, sem)
            for r in range(C):
                row = cidx * C + r
                row_splat = jnp.full((L,), row, jnp.int32)
                noffs = tuple(
                    (negids_v[cidx, pl.ds(r * NUM_SAMPLED + grp * L, L)] & 1)
                    * D_MODEL
                    for grp in range(NGRP))

                @pl.loop(0, D_MODEL, init_carry=zero8, unroll=2)
                def _neg_dots(d, accs, nbuf=nbuf, r=r, row_splat=row_splat,
                              noffs=noffs):
                    t = d & (L - 1)
                    dcol = (d - t) + ((t + iot) & (L - 1))
                    u = plsc.load_gather(user_v, [row_splat, dcol])
                    return tuple(
                        accs[grp]
                        + u * plsc.load_gather(
                            nbuf,
                            [r * NUM_SAMPLED + grp * L + iot,
                             noffs[grp] + dcol])
                        for grp in range(NGRP)
                    )

                accs = _neg_dots
                nm = accs[0]
                for grp in range(1, NGRP):
                    nm = jnp.maximum(nm, accs[grp])
                ps_v = plsc.load_gather(pos_v, [row_splat])
                ps = jnp.max(ps_v)
                m = jnp.maximum(jnp.max(nm), ps)
                s = jnp.where(lane0, jnp.exp(ps_v - m), 0.0)
                for grp in range(NGRP):
                    s = s + jnp.exp(accs[grp] - m)
                se = jnp.sum(s)
                mp = m - ps
                plsc.store_scatter(se_v, [row_splat], jnp.full((L,), se),
                                   mask=lane0)
                plsc.store_scatter(mp_v, [row_splat], jnp.full((L,), mp),
                                   mask=lane0)

            @pl.when(cidx + 2 < NCHUNK)
            def _issue_next(cidx=cidx, nbuf=nbuf, sem=sem):
                _issue(cidx + 2, nbuf, sem)

    pltpu.sync_copy(se_v, se_out.at[pl.ds(base, BPW)])
    pltpu.sync_copy(mp_v, mp_out.at[pl.ds(base, BPW)])


NBLK = NUM_CLASSES // 128
TAIL = NUM_CLASSES - NBLK * 128
NJ = 246


@functools.partial(
    pl.kernel,
    out_type=jax.ShapeDtypeStruct((TPAIR, 2 * D_MODEL), jnp.float32),
    mesh=_mesh,
    scratch_types=(
        pltpu.VMEM((D_MODEL, 128), jnp.float32),
        pltpu.VMEM((D_MODEL, 128), jnp.float32),
        pltpu.VMEM((64, 128), jnp.float32),
        pltpu.VMEM((64, 128), jnp.float32),
        pltpu.SemaphoreType.DMA,
        pltpu.SemaphoreType.DMA,
        pltpu.SemaphoreType.DMA,
        pltpu.SemaphoreType.DMA,
    ),
    compiler_params=pltpu.CompilerParams(needs_layout_passes=False,
                                         use_tc_tiling_on_sc=True),
)
def _transpose_table(tT_hbm, tp_out, in_a, in_b, out_a, out_b,
                     sem_ia, sem_ib, sem_oa, sem_ob):
    wid = lax.axis_index("s") * NC + lax.axis_index("c")
    iot = lax.iota(jnp.int32, L)
    dperm = tuple((iot + rot) & (L - 1) for rot in range(L))

    def issue_in(ib, in_v, sem):
        for s in range(D_MODEL // 8):
            pltpu.async_copy(tT_hbm.at[pl.ds(s * 8, 8), pl.ds(ib * 128, 128)],
                             in_v.at[pl.ds(s * 8, 8)], sem)

    def drain_in(in_v, sem):
        for s in range(D_MODEL // 8):
            pltpu.make_async_copy(
                tT_hbm.at[pl.ds(0, 8), pl.ds(0, 128)],
                in_v.at[pl.ds(s * 8, 8)], sem).wait()

    def shuffle(in_v, out_v, width):
        @pl.loop(0, width, step=L)
        def _cols(iloc0):
            iloc_vec = iloc0 + iot
            pp = iloc_vec >> 1
            cbase = (iloc_vec & 1) * D_MODEL
            for d0 in range(0, D_MODEL, L):
                for rot in range(L):
                    d_vec = d0 + dperm[rot]
                    vals = plsc.load_gather(in_v, [d_vec, iloc_vec])
                    plsc.store_scatter(out_v, [pp, cbase + d_vec], vals)

    def drain_out(out_v, sem):
        pltpu.make_async_copy(out_v, tp_out.at[pl.ds(0, 64)], sem).wait()

    issue_in(wid, in_a, sem_ia)
    issue_in(wid + NW, in_b, sem_ib)

    @pl.loop(0, NJ, step=2)
    def _blocks(j):
        for buf, (in_v, out_v, sem_i, sem_o) in enumerate(
                ((in_a, out_a, sem_ia, sem_oa),
                 (in_b, out_b, sem_ib, sem_ob))):
            jj = j + buf
            ib = wid + NW * jj

            @pl.when(ib < NBLK)
            def _do(jj=jj, ib=ib, in_v=in_v, out_v=out_v,
                    sem_i=sem_i, sem_o=sem_o):
                drain_in(in_v, sem_i)

                @pl.when(jj >= 2)
                def _wait_prev_out():
                    drain_out(out_v, sem_o)

                shuffle(in_v, out_v, 128)
                pltpu.async_copy(out_v, tp_out.at[pl.ds(ib * 64, 64)], sem_o)

                @pl.when(ib + 2 * NW < NBLK)
                def _issue_next():
                    issue_in(ib + 2 * NW, in_v, sem_i)

    drain_out(out_a, sem_oa)
    drain_out(out_b, sem_ob)

    @pl.when(wid == NW - 1)
    def _tail():
        for d in range(D_MODEL):
            pltpu.async_copy(
                tT_hbm.at[d, pl.ds(NBLK * 128, TAIL)],
                in_a.at[d, pl.ds(0, TAIL)], sem_ia)
        for d in range(D_MODEL):
            pltpu.make_async_copy(
                tT_hbm.at[0, pl.ds(0, TAIL)],
                in_a.at[d, pl.ds(0, TAIL)], sem_ia).wait()
        shuffle(in_a, out_a, TAIL)
        pltpu.sync_copy(out_a.at[pl.ds(0, TAIL // 2)],
                        tp_out.at[pl.ds(NBLK * 64, TAIL // 2)])


def _tc_finish_body(se_ref, mp_ref, o_ref):
    x = jnp.log(se_ref[...]) + mp_ref[...]
    o_ref[...] = jnp.reshape(jnp.sum(x) * (1.0 / BATCH), (1, 1))


_tc_finish = pl.pallas_call(
    _tc_finish_body,
    out_shape=jax.ShapeDtypeStruct((1, 1), jnp.float32),
)


def kernel(user_embeddings, item_embeddings, positive_item_ids,
           negative_item_ids):
    pos_ids = positive_item_ids.astype(jnp.int32)
    neg_ids = negative_item_ids.astype(jnp.int32).reshape(
        NW, NCHUNK, C * NUM_SAMPLED)
    table_pairs = _transpose_table(item_embeddings.T)
    se, mp = _sc_sampled_softmax(
        user_embeddings, table_pairs, pos_ids, neg_ids)
    loss = _tc_finish(se.reshape(NW, BPW), mp.reshape(NW, BPW))
    return loss[0, 0]


# speedup vs baseline: 1.2504x; 1.2504x over previous
import functools

import jax
import jax.numpy as jnp
from jax import lax
from jax.experimental import pallas as pl
from jax.experimental.pallas import tpu as pltpu
from jax.experimental.pallas import tpu_sc as plsc

NUM_CLASSES = 1000000
NUM_SAMPLED = 128
BATCH = 4096
D_MODEL = 64

NC, NS, L = 2, 16, 16
NW = NC * NS
BPW = BATCH // NW
C = 4
NCHUNK = BPW // C
NGRP = NUM_SAMPLED // L

_mesh = plsc.VectorSubcoreMesh(
    core_axis_name="c", subcore_axis_name="s", num_cores=NC, num_subcores=NS)


@functools.partial(
    pl.kernel,
    out_type=(
        jax.ShapeDtypeStruct((BATCH,), jnp.float32),
        jax.ShapeDtypeStruct((BATCH,), jnp.float32),
    ),
    mesh=_mesh,
    scratch_types=(
        pltpu.VMEM((BPW, D_MODEL), jnp.float32),
        pltpu.VMEM((BPW, D_MODEL), jnp.float32),
        pltpu.VMEM((BPW, NUM_SAMPLED), jnp.int32),
        pltpu.VMEM((BPW,), jnp.int32),
        pltpu.VMEM((C, NUM_SAMPLED, D_MODEL), jnp.float32),
        pltpu.VMEM((C, NUM_SAMPLED, D_MODEL), jnp.float32),
        pltpu.VMEM((BPW,), jnp.float32),
        pltpu.VMEM((BPW,), jnp.float32),
        pltpu.VMEM((BPW,), jnp.float32),
        pltpu.SemaphoreType.DMA,
        pltpu.SemaphoreType.DMA,
        pltpu.SemaphoreType.DMA,
    ),
    compiler_params=pltpu.CompilerParams(needs_layout_passes=False,
                                         use_tc_tiling_on_sc=False),
)
def _sc_sampled_softmax(user_hbm, table_hbm, posid_hbm, negid_hbm,
                        se_out, mp_out,
                        user_v, posrows_v, negids_v, posids_v,
                        nbuf_a, nbuf_b, pos_v, se_v, mp_v,
                        sem_p, sem_a, sem_b):
    wid = lax.axis_index("s") * NC + lax.axis_index("c")
    base = wid * BPW
    iot = lax.iota(jnp.int32, L)
    lane0 = iot == 0

    pltpu.sync_copy(posid_hbm.at[pl.ds(base, BPW)], posids_v)
    pltpu.sync_copy(negid_hbm.at[pl.ds(base, BPW)], negids_v)
    pltpu.sync_copy(user_hbm.at[pl.ds(base, BPW)], user_v)

    def _issue(c, nbuf, sem):
        for r in range(C):
            for k in range(NUM_SAMPLED // L):
                idxv = negids_v[c * C + r, pl.ds(k * L, L)]
                pltpu.async_copy(table_hbm.at[idxv],
                                 nbuf.at[r].at[pl.ds(k * L, L)], sem)

    def _drain(c, nbuf, sem):
        for r in range(C):
            for k in range(NUM_SAMPLED // L):
                pltpu.make_async_copy(table_hbm.at[pl.ds(0, L)],
                                      nbuf.at[r].at[pl.ds(k * L, L)],
                                      sem).wait()

    for k in range(BPW // L):
        pidxv = posids_v[pl.ds(k * L, L)]
        pltpu.async_copy(table_hbm.at[pidxv],
                         posrows_v.at[pl.ds(k * L, L)], sem_p)
    _issue(0, nbuf_a, sem_a)
    _issue(1, nbuf_b, sem_b)
    for k in range(BPW // L):
        pltpu.make_async_copy(table_hbm.at[pl.ds(0, L)],
                              posrows_v.at[pl.ds(k * L, L)], sem_p).wait()

    for grp in range(BPW // L):
        rows = grp * L + iot

        @pl.loop(0, D_MODEL, init_carry=jnp.zeros((L,), jnp.float32), unroll=4)
        def _pos_dot(d, acc, rows=rows):
            t = d & (L - 1)
            dcol = (d - t) + ((t + iot) & (L - 1))
            u = plsc.load_gather(user_v, [rows, dcol])
            p = plsc.load_gather(posrows_v, [rows, dcol])
            return acc + u * p

        pos_v[pl.ds(grp * L, L)] = _pos_dot

    zero8 = tuple(jnp.zeros((L,), jnp.float32) for _ in range(NGRP))

    @pl.loop(0, NCHUNK, step=2)
    def _chunks(g):
        for buf, (nbuf, sem) in enumerate(((nbuf_a, sem_a), (nbuf_b, sem_b))):
            cidx = g + buf
            _drain(cidx, nbuf, sem)
            for r in range(C):
                row = cidx * C + r
                row_splat = jnp.full((L,), row, jnp.int32)

                @pl.loop(0, D_MODEL, init_carry=zero8, unroll=2)
                def _neg_dots(d, accs, nref=nbuf.at[r], row_splat=row_splat):
                    t = d & (L - 1)
                    dcol = (d - t) + ((t + iot) & (L - 1))
                    u = plsc.load_gather(user_v, [row_splat, dcol])
                    return tuple(
                        accs[grp]
                        + u * plsc.load_gather(nref, [grp * L + iot, dcol])
                        for grp in range(NGRP)
                    )

                accs = _neg_dots
                nm = accs[0]
                for grp in range(1, NGRP):
                    nm = jnp.maximum(nm, accs[grp])
                ps_v = plsc.load_gather(pos_v, [row_splat])
                ps = jnp.max(ps_v)
                m = jnp.maximum(jnp.max(nm), ps)
                s = jnp.where(lane0, jnp.exp(ps_v - m), 0.0)
                for grp in range(NGRP):
                    s = s + jnp.exp(accs[grp] - m)
                se = jnp.sum(s)
                mp = m - ps
                plsc.store_scatter(se_v, [row_splat], jnp.full((L,), se),
                                   mask=lane0)
                plsc.store_scatter(mp_v, [row_splat], jnp.full((L,), mp),
                                   mask=lane0)

            @pl.when(cidx + 2 < NCHUNK)
            def _issue_next(cidx=cidx, nbuf=nbuf, sem=sem):
                _issue(cidx + 2, nbuf, sem)

    pltpu.sync_copy(se_v, se_out.at[pl.ds(base, BPW)])
    pltpu.sync_copy(mp_v, mp_out.at[pl.ds(base, BPW)])


def _tc_finish_body(se_ref, mp_ref, o_ref):
    x = jnp.log(se_ref[...]) + mp_ref[...]
    o_ref[...] = jnp.reshape(jnp.sum(x) * (1.0 / BATCH), (1, 1))


_tc_finish = pl.pallas_call(
    _tc_finish_body,
    out_shape=jax.ShapeDtypeStruct((1, 1), jnp.float32),
)


def kernel(user_embeddings, item_embeddings, positive_item_ids,
           negative_item_ids):
    pos_ids = positive_item_ids.astype(jnp.int32)
    neg_ids = negative_item_ids.astype(jnp.int32)
    se, mp = _sc_sampled_softmax(
        user_embeddings, item_embeddings, pos_ids, neg_ids)
    loss = _tc_finish(se.reshape(NW, BPW), mp.reshape(NW, BPW))
    return loss[0, 0]
